# baseline (speedup 1.0000x reference)
import functools

import jax
import jax.numpy as jnp
import numpy as np
from jax import lax
from jax.experimental import pallas as pl
from jax.experimental.pallas import tpu as pltpu
from jax.experimental.pallas import tpu_sc as plsc

_TEMPERATURE = 1.0
_BIAS = 0.0001
_L = 16


@functools.lru_cache(maxsize=None)
def _gate_noise(e):
    with jax.ensure_compile_time_eval(), jax.default_device(jax.devices("cpu")[0]):
        u = jax.random.uniform(jax.random.key(1), (e,), dtype=jnp.float32)
        eps = (_BIAS - (1.0 - _BIAS)) * u + (1.0 - _BIAS)
        g = (jnp.log(eps) - jnp.log(1.0 - eps)) / _TEMPERATURE
        return np.asarray(jax.block_until_ready(g))


def _tc_body(emb_ref, w1_ref, b1_ref, we_ref, be_ref, p_ref):
    h_dim = w1_ref.shape[1]
    x = jnp.dot(emb_ref[...], w1_ref[...], preferred_element_type=jnp.float32)
    h = jnp.maximum(x + b1_ref[...], 0.0)
    c = (we_ref[:h_dim, :] + we_ref[h_dim:, :]) * 0.5
    p_ref[...] = jnp.dot(h, c, preferred_element_type=jnp.float32) + 0.5 * be_ref[0, 0]


def _make_sc_kernel(n_nodes, n_edges, nw, unroll):
    blocks = n_edges // 128
    nb_main = -(-blocks // nw)
    nb_last = blocks - (nw - 1) * nb_main
    epw_max = nb_main * 128
    mesh = plsc.VectorSubcoreMesh(core_axis_name="c", subcore_axis_name="s")

    @functools.partial(
        pl.kernel,
        mesh=mesh,
        out_type=jax.ShapeDtypeStruct((n_edges,), jnp.float32),
        compiler_params=pltpu.CompilerParams(needs_layout_passes=False),
        scratch_types=[
            pltpu.VMEM((n_nodes,), jnp.float32),
            pltpu.VMEM((2, epw_max), jnp.int32),
            pltpu.VMEM((epw_max,), jnp.float32),
            pltpu.VMEM((epw_max,), jnp.float32),
            pltpu.SemaphoreType.DMA,
        ],
    )
    def sc_edge_gate(p_hbm, edges_hbm, g_hbm, out_hbm,
                     p_v, ed_v, g_v, o_v, sem):
        wid = lax.axis_index("s") * 2 + lax.axis_index("c")
        h_p = pltpu.async_copy(p_hbm, p_v, sem)

        def run(bstart, nb):
            es = bstart * 128
            esz = nb * 128
            h_e = pltpu.async_copy(edges_hbm.at[:, pl.ds(es, esz)],
                                   ed_v.at[:, pl.ds(0, esz)], sem)
            h_g = pltpu.async_copy(g_hbm.at[pl.ds(es, esz)],
                                   g_v.at[pl.ds(0, esz)], sem)
            h_p.wait()
            h_e.wait()
            h_g.wait()

            @plsc.parallel_loop(0, nb, unroll=unroll)
            def _(b):
                for k in range(8):
                    sl = pl.ds(b * 128 + k * _L, _L)
                    vs = plsc.load_gather(p_v, [ed_v[0, sl]])
                    vd = plsc.load_gather(p_v, [ed_v[1, sl]])
                    x = vs + vd + g_v[sl]
                    o_v[sl] = 1.0 / (1.0 + jnp.exp(-x))

            pltpu.sync_copy(o_v.at[pl.ds(0, esz)],
                            out_hbm.at[pl.ds(es, esz)])

        @pl.when(wid < nw - 1)
        def _():
            run(jnp.minimum(wid, nw - 2) * nb_main, nb_main)

        @pl.when(wid == nw - 1)
        def _():
            run((nw - 1) * nb_main, nb_last)

    return sc_edge_gate


def kernel(embedding, edges, W1, b1, We, be):
    n, d = embedding.shape
    h_dim = W1.shape[1]
    e = edges.shape[1]

    sc_call = _make_sc_kernel(n, e, 32, unroll=4)
    g = jnp.asarray(_gate_noise(e))

    p2d = pl.pallas_call(
        _tc_body,
        grid=(1,),
        in_specs=[
            pl.BlockSpec((n, d), lambda i: (0, 0)),
            pl.BlockSpec((d, h_dim), lambda i: (0, 0)),
            pl.BlockSpec((1, h_dim), lambda i: (0, 0)),
            pl.BlockSpec((2 * h_dim, 1), lambda i: (0, 0)),
            pl.BlockSpec((1, 1), lambda i: (0, 0)),
        ],
        out_specs=pl.BlockSpec((n, 1), lambda i: (0, 0)),
        out_shape=jax.ShapeDtypeStruct((n, 1), jnp.float32),
    )(embedding, W1, b1.reshape(1, h_dim), We, be.reshape(1, 1))

    p = p2d.reshape(n)
    return sc_call(p, edges, g)


# speedup vs baseline: 1.0054x; 1.0054x over previous
import functools

import jax
import jax.numpy as jnp
import numpy as np
from jax import lax
from jax.experimental import pallas as pl
from jax.experimental.pallas import tpu as pltpu
from jax.experimental.pallas import tpu_sc as plsc

_TEMPERATURE = 1.0
_BIAS = 0.0001
_L = 16


@functools.lru_cache(maxsize=None)
def _gate_noise(e):
    with jax.ensure_compile_time_eval(), jax.default_device(jax.devices("cpu")[0]):
        u = jax.random.uniform(jax.random.key(1), (e,), dtype=jnp.float32)
        eps = (_BIAS - (1.0 - _BIAS)) * u + (1.0 - _BIAS)
        g = (jnp.log(eps) - jnp.log(1.0 - eps)) / _TEMPERATURE
        return np.asarray(jax.block_until_ready(g))


def _tc_body(emb_ref, w1_ref, b1_ref, we_ref, be_ref, p_ref):
    h_dim = w1_ref.shape[1]
    x = jnp.dot(emb_ref[...], w1_ref[...], preferred_element_type=jnp.float32)
    h = jnp.maximum(x + b1_ref[...], 0.0)
    c = (we_ref[:h_dim, :] + we_ref[h_dim:, :]) * 0.5
    p_ref[...] = jnp.dot(h, c, preferred_element_type=jnp.float32) + 0.5 * be_ref[0, 0]


def _make_sc_kernel(n_nodes, n_edges, nw, unroll):
    blocks = n_edges // 128
    nb_main = -(-blocks // nw)
    nb_last = blocks - (nw - 1) * nb_main
    epw_max = nb_main * 128
    mesh = plsc.VectorSubcoreMesh(core_axis_name="c", subcore_axis_name="s")

    @functools.partial(
        pl.kernel,
        mesh=mesh,
        out_type=jax.ShapeDtypeStruct((n_edges,), jnp.float32),
        compiler_params=pltpu.CompilerParams(needs_layout_passes=False),
        scratch_types=[
            pltpu.VMEM((n_nodes,), jnp.float32),
            pltpu.VMEM((2, epw_max), jnp.int32),
            pltpu.VMEM((epw_max,), jnp.float32),
            pltpu.VMEM((epw_max,), jnp.float32),
            pltpu.SemaphoreType.DMA,
        ],
    )
    def sc_edge_gate(p_hbm, edges_hbm, g_hbm, out_hbm,
                     p_v, ed_v, g_v, o_v, sem):
        wid = lax.axis_index("s") * 2 + lax.axis_index("c")
        h_p = pltpu.async_copy(p_hbm, p_v, sem)

        def run(bstart, nb):
            es = bstart * 128
            esz = nb * 128
            h_e = pltpu.async_copy(edges_hbm.at[:, pl.ds(es, esz)],
                                   ed_v.at[:, pl.ds(0, esz)], sem)
            h_g = pltpu.async_copy(g_hbm.at[pl.ds(es, esz)],
                                   g_v.at[pl.ds(0, esz)], sem)
            h_p.wait()
            h_e.wait()
            h_g.wait()

            @plsc.parallel_loop(0, nb, unroll=unroll)
            def _(b):
                for k in range(8):
                    sl = pl.ds(b * 128 + k * _L, _L)
                    vs = plsc.load_gather(p_v, [ed_v[0, sl]])
                    vd = plsc.load_gather(p_v, [ed_v[1, sl]])
                    x = vs + vd + g_v[sl]
                    o_v[sl] = 1.0 / (1.0 + jnp.exp(-x))

            pltpu.sync_copy(o_v.at[pl.ds(0, esz)],
                            out_hbm.at[pl.ds(es, esz)])

        @pl.when(wid < nw - 1)
        def _():
            run(jnp.minimum(wid, nw - 2) * nb_main, nb_main)

        @pl.when(wid == nw - 1)
        def _():
            run((nw - 1) * nb_main, nb_last)

    return sc_edge_gate


def kernel(embedding, edges, W1, b1, We, be):
    n, d = embedding.shape
    h_dim = W1.shape[1]
    e = edges.shape[1]

    sc_call = _make_sc_kernel(n, e, 32, unroll=2)
    g = jnp.asarray(_gate_noise(e))

    p2d = pl.pallas_call(
        _tc_body,
        grid=(1,),
        in_specs=[
            pl.BlockSpec((n, d), lambda i: (0, 0)),
            pl.BlockSpec((d, h_dim), lambda i: (0, 0)),
            pl.BlockSpec((1, h_dim), lambda i: (0, 0)),
            pl.BlockSpec((2 * h_dim, 1), lambda i: (0, 0)),
            pl.BlockSpec((1, 1), lambda i: (0, 0)),
        ],
        out_specs=pl.BlockSpec((n, 1), lambda i: (0, 0)),
        out_shape=jax.ShapeDtypeStruct((n, 1), jnp.float32),
    )(embedding, W1, b1.reshape(1, h_dim), We, be.reshape(1, 1))

    p = p2d.reshape(n)
    return sc_call(p, edges, g)


# speedup vs baseline: 1.0246x; 1.0191x over previous
import functools

import jax
import jax.numpy as jnp
import numpy as np
from jax import lax
from jax.experimental import pallas as pl
from jax.experimental.pallas import tpu as pltpu
from jax.experimental.pallas import tpu_sc as plsc

_TEMPERATURE = 1.0
_BIAS = 0.0001
_L = 16


@functools.lru_cache(maxsize=None)
def _gate_noise(e):
    with jax.ensure_compile_time_eval(), jax.default_device(jax.devices("cpu")[0]):
        u = jax.random.uniform(jax.random.key(1), (e,), dtype=jnp.float32)
        eps = (_BIAS - (1.0 - _BIAS)) * u + (1.0 - _BIAS)
        g = (jnp.log(eps) - jnp.log(1.0 - eps)) / _TEMPERATURE
        return np.asarray(jax.block_until_ready(g))


def _tc_body(emb_ref, w1_ref, b1_ref, we_ref, be_ref, p_ref):
    h_dim = w1_ref.shape[1]
    x = jnp.dot(emb_ref[...], w1_ref[...], preferred_element_type=jnp.float32)
    h = jnp.maximum(x + b1_ref[...], 0.0)
    c = (we_ref[:h_dim, :] + we_ref[h_dim:, :]) * 0.5
    p_ref[...] = jnp.dot(h, c, preferred_element_type=jnp.float32) + 0.5 * be_ref[0, 0]


def _make_sc_kernel(n_nodes, n_edges, nw, unroll):
    blocks = n_edges // 128
    nb_main = -(-blocks // nw)
    nb_last = blocks - (nw - 1) * nb_main
    epw_max = nb_main * 128
    mesh = plsc.VectorSubcoreMesh(core_axis_name="c", subcore_axis_name="s")

    @functools.partial(
        pl.kernel,
        mesh=mesh,
        out_type=jax.ShapeDtypeStruct((n_edges,), jnp.float32),
        compiler_params=pltpu.CompilerParams(needs_layout_passes=False),
        scratch_types=[
            pltpu.VMEM((n_nodes,), jnp.float32),
            pltpu.VMEM((2, epw_max), jnp.int32),
            pltpu.VMEM((epw_max,), jnp.float32),
            pltpu.VMEM((epw_max,), jnp.float32),
            pltpu.SemaphoreType.DMA,
        ],
    )
    def sc_edge_gate(p_hbm, edges_hbm, g_hbm, out_hbm,
                     p_v, ed_v, g_v, o_v, sem):
        wid = lax.axis_index("s") * 2 + lax.axis_index("c")
        h_p = pltpu.async_copy(p_hbm, p_v, sem)

        def run(bstart, nb):
            es = bstart * 128
            esz = nb * 128
            h_e = pltpu.async_copy(edges_hbm.at[:, pl.ds(es, esz)],
                                   ed_v.at[:, pl.ds(0, esz)], sem)
            h_g = pltpu.async_copy(g_hbm.at[pl.ds(es, esz)],
                                   g_v.at[pl.ds(0, esz)], sem)
            h_p.wait()
            h_e.wait()
            h_g.wait()

            @plsc.parallel_loop(0, nb, unroll=unroll)
            def _(b):
                for k in range(8):
                    sl = pl.ds(b * 128 + k * _L, _L)
                    vs = plsc.load_gather(p_v, [ed_v[0, sl]])
                    vd = plsc.load_gather(p_v, [ed_v[1, sl]])
                    x = vs + vd + g_v[sl]
                    o_v[sl] = 1.0 / (1.0 + jnp.exp(-x))

            pltpu.sync_copy(o_v.at[pl.ds(0, esz)],
                            out_hbm.at[pl.ds(es, esz)])

        @pl.when(wid < nw - 1)
        def _():
            run(jnp.minimum(wid, nw - 2) * nb_main, nb_main)

        @pl.when(wid == nw - 1)
        def _():
            run((nw - 1) * nb_main, nb_last)

    return sc_edge_gate


def kernel(embedding, edges, W1, b1, We, be):
    n, d = embedding.shape
    h_dim = W1.shape[1]
    e = edges.shape[1]

    sc_call = _make_sc_kernel(n, e, 32, unroll=2)
    g = jnp.asarray(_gate_noise(e))

    bn = 2000
    p2d = pl.pallas_call(
        _tc_body,
        grid=(n // bn,),
        in_specs=[
            pl.BlockSpec((bn, d), lambda i: (i, 0)),
            pl.BlockSpec((d, h_dim), lambda i: (0, 0)),
            pl.BlockSpec((1, h_dim), lambda i: (0, 0)),
            pl.BlockSpec((2 * h_dim, 1), lambda i: (0, 0)),
            pl.BlockSpec((1, 1), lambda i: (0, 0)),
        ],
        out_specs=pl.BlockSpec((bn, 1), lambda i: (i, 0)),
        out_shape=jax.ShapeDtypeStruct((n, 1), jnp.float32),
        compiler_params=pltpu.CompilerParams(vmem_limit_bytes=4 * 1024 * 1024),
    )(embedding, W1, b1.reshape(1, h_dim), We, be.reshape(1, 1))

    p = p2d.reshape(n)
    return sc_call(p, edges, g)
